# Initial kernel scaffold; baseline (speedup 1.0000x reference)
import jax
import jax.numpy as jnp
from jax.experimental import pallas as pl


def kernel(x, edge_index, batch, W0, a_src0, a_dst0, b0, g0, be0, W1, a_src1, a_dst1, b1, g1, be1, dW1, db1, dW2, db2):
    raise NotImplementedError("write your pallas kernel here")


# speedup vs baseline: 23.5601x; 23.5601x over previous
import jax
import jax.numpy as jnp
from jax import lax
from jax.experimental import pallas as pl
from jax.experimental.pallas import tpu as pltpu
from jax.experimental.pallas import tpu_sc as plsc

N = 10000
E = 320000
D = 128
C = 128
G = 64

NPAD = 10240
HALF = NPAD // 2
ROW = 144
NS = 16
ECH = 20672
EPAD = NS * ECH
NG = ECH // 16
BLK = 1024
NRT = HALF // NS


def _lrelu(z):
    return jnp.maximum(z, 0.2 * z)


def _attn_heads(th, asv, adv, as_ref, ad_ref, c_ref):
    asr = jnp.sum(th * asv, axis=1, keepdims=True)
    adr = jnp.sum(th * adv, axis=1, keepdims=True)
    as_ref[...] = asr
    ad_ref[...] = adr
    c_ref[...] = _lrelu(asr + adr)


def _ones_col(n):
    lane = lax.broadcasted_iota(jnp.int32, (n, ROW - C), 1)
    return jnp.where(lane == 0, 1.0, 0.0).astype(jnp.float32)


def _tc_prep_body(x_ref, w_ref, asv_ref, adv_ref, ext_ref, as_ref, ad_ref, c_ref):
    th = jnp.dot(x_ref[...], w_ref[...], preferred_element_type=jnp.float32)
    _attn_heads(th, asv_ref[...], adv_ref[...], as_ref, ad_ref, c_ref)
    ext_ref[...] = jnp.concatenate([th, _ones_col(BLK)], axis=1)


def _combine_ln(acc_ref, b_ref, g_ref, be_ref):
    s = acc_ref[...]
    o = s[:, :C] / (s[:, C:C + 1] + 1e-16) + b_ref[...]
    m = jnp.mean(o, axis=1, keepdims=True)
    v = jnp.mean((o - m) * (o - m), axis=1, keepdims=True)
    hn = (o - m) / jnp.sqrt(v + 1e-5) * g_ref[...] + be_ref[...]
    return jnp.maximum(hn, 0.0)


def _tc_mid_body(acc_ref, b_ref, g_ref, be_ref, w_ref, asv_ref, adv_ref,
                 ext_ref, as_ref, ad_ref, c_ref):
    h = _combine_ln(acc_ref, b_ref, g_ref, be_ref)
    th = jnp.dot(h, w_ref[...], preferred_element_type=jnp.float32)
    _attn_heads(th, asv_ref[...], adv_ref[...], as_ref, ad_ref, c_ref)
    ext_ref[...] = jnp.concatenate([th, _ones_col(BLK)], axis=1)


def _tc_final_body(acc_ref, b_ref, g_ref, be_ref, batch_ref, dw1_ref, db1_ref,
                   dw2_ref, db2_ref, pool_ref, out_ref):
    i = pl.program_id(0)
    h = _combine_ln(acc_ref, b_ref, g_ref, be_ref)
    ext = jnp.concatenate([h, _ones_col(BLK)], axis=1)
    gid = lax.broadcasted_iota(jnp.int32, (1, G), 1)
    oh = (batch_ref[...] == gid).astype(jnp.float32)
    part = lax.dot_general(oh, ext, (((0,), (0,)), ((), ())),
                           preferred_element_type=jnp.float32)

    @pl.when(i == 0)
    def _():
        pool_ref[...] = jnp.zeros_like(pool_ref)

    pool_ref[...] += part

    @pl.when(i == NPAD // BLK - 1)
    def _():
        p = pool_ref[...]
        pooled = p[:, :C] / jnp.maximum(p[:, C:C + 1], 1.0)
        t = jnp.dot(pooled, dw1_ref[...], preferred_element_type=jnp.float32)
        t = jnp.maximum(t + db1_ref[...], 0.0)
        out_ref[...] = jnp.dot(t, dw2_ref[...],
                               preferred_element_type=jnp.float32) + db2_ref[...]


def _tc_prep(xp, w, asv, adv):
    return pl.pallas_call(
        _tc_prep_body,
        grid=(NPAD // BLK,),
        in_specs=[
            pl.BlockSpec((BLK, D), lambda i: (i, 0)),
            pl.BlockSpec((D, C), lambda i: (0, 0)),
            pl.BlockSpec((1, C), lambda i: (0, 0)),
            pl.BlockSpec((1, C), lambda i: (0, 0)),
        ],
        out_specs=[
            pl.BlockSpec((BLK, ROW), lambda i: (i, 0)),
            pl.BlockSpec((BLK, 1), lambda i: (i, 0)),
            pl.BlockSpec((BLK, 1), lambda i: (i, 0)),
            pl.BlockSpec((BLK, 1), lambda i: (i, 0)),
        ],
        out_shape=[
            jax.ShapeDtypeStruct((NPAD, ROW), jnp.float32),
            jax.ShapeDtypeStruct((NPAD, 1), jnp.float32),
            jax.ShapeDtypeStruct((NPAD, 1), jnp.float32),
            jax.ShapeDtypeStruct((NPAD, 1), jnp.float32),
        ],
    )(xp, w, asv, adv)


def _tc_mid(acc, b, g, be, w, asv, adv):
    return pl.pallas_call(
        _tc_mid_body,
        grid=(NPAD // BLK,),
        in_specs=[
            pl.BlockSpec((BLK, ROW), lambda i: (i, 0)),
            pl.BlockSpec((1, C), lambda i: (0, 0)),
            pl.BlockSpec((1, C), lambda i: (0, 0)),
            pl.BlockSpec((1, C), lambda i: (0, 0)),
            pl.BlockSpec((C, C), lambda i: (0, 0)),
            pl.BlockSpec((1, C), lambda i: (0, 0)),
            pl.BlockSpec((1, C), lambda i: (0, 0)),
        ],
        out_specs=[
            pl.BlockSpec((BLK, ROW), lambda i: (i, 0)),
            pl.BlockSpec((BLK, 1), lambda i: (i, 0)),
            pl.BlockSpec((BLK, 1), lambda i: (i, 0)),
            pl.BlockSpec((BLK, 1), lambda i: (i, 0)),
        ],
        out_shape=[
            jax.ShapeDtypeStruct((NPAD, ROW), jnp.float32),
            jax.ShapeDtypeStruct((NPAD, 1), jnp.float32),
            jax.ShapeDtypeStruct((NPAD, 1), jnp.float32),
            jax.ShapeDtypeStruct((NPAD, 1), jnp.float32),
        ],
    )(acc, b, g, be, w, asv, adv)


def _tc_final(acc, b, g, be, batch2d, dw1, db1, dw2, db2):
    return pl.pallas_call(
        _tc_final_body,
        grid=(NPAD // BLK,),
        in_specs=[
            pl.BlockSpec((BLK, ROW), lambda i: (i, 0)),
            pl.BlockSpec((1, C), lambda i: (0, 0)),
            pl.BlockSpec((1, C), lambda i: (0, 0)),
            pl.BlockSpec((1, C), lambda i: (0, 0)),
            pl.BlockSpec((BLK, 1), lambda i: (i, 0)),
            pl.BlockSpec((C, 2 * C), lambda i: (0, 0)),
            pl.BlockSpec((1, 2 * C), lambda i: (0, 0)),
            pl.BlockSpec((2 * C, C), lambda i: (0, 0)),
            pl.BlockSpec((1, C), lambda i: (0, 0)),
        ],
        out_specs=[
            pl.BlockSpec((G, ROW), lambda i: (0, 0)),
            pl.BlockSpec((G, C), lambda i: (0, 0)),
        ],
        out_shape=[
            jax.ShapeDtypeStruct((G, ROW), jnp.float32),
            jax.ShapeDtypeStruct((G, C), jnp.float32),
        ],
    )(acc, b, g, be, batch2d, dw1, db1, dw2, db2)


def _sc_edge_body(theta_ref, asrc_ref, adst_ref, cst_ref, src_ref, dst_ref,
                  out_ref, asrc_v, adst_v, cst_v, src_v, dst_v, rows0, rows1,
                  acc, gsem0, gsem1, ssem):
    cid = lax.axis_index("c")
    sid = lax.axis_index("s")
    ebase = sid * ECH
    rlo = cid * HALF

    pltpu.sync_copy(asrc_ref, asrc_v)
    pltpu.sync_copy(adst_ref, adst_v)
    pltpu.sync_copy(cst_ref, cst_v)
    pltpu.sync_copy(src_ref.at[pl.ds(ebase, ECH)], src_v.at[pl.ds(0, ECH)])
    pltpu.sync_copy(dst_ref.at[pl.ds(ebase, ECH)], dst_v.at[pl.ds(0, ECH)])

    zero = jnp.zeros((16,), jnp.float32)
    for j in range(16):
        for cc in range(ROW // 16):
            rows0[j, pl.ds(cc * 16, 16)] = zero
    lo = sid * NRT
    for k in range(NRT // 16):
        pltpu.sync_copy(rows0, acc.at[pl.ds(lo + k * 16, 16)])
    plsc.subcore_barrier()

    def _compact(g, cnt):
        off = g * 16
        s16 = src_v[pl.ds(off, 16)]
        d16 = dst_v[pl.ds(off, 16)]
        keep = (d16 >= rlo) & (d16 < rlo + HALF)
        plsc.store_compressed(src_v.at[pl.ds(cnt, 16)], s16, mask=keep)
        plsc.store_compressed(dst_v.at[pl.ds(cnt, 16)], d16, mask=keep)
        return cnt + plsc.all_reduce_population_count(keep)[0]

    cnt = lax.fori_loop(0, NG, _compact, jnp.int32(0))

    lane = lax.iota(jnp.int32, 16)

    def _indices(g):
        off = g * 16
        m = lane < (cnt - off)
        s16 = jnp.where(m, src_v[pl.ds(off, 16)], 0)
        d16 = jnp.where(m, dst_v[pl.ds(off, 16)], rlo)
        return m, s16, d16

    _, s16a, _ = _indices(jnp.int32(0))
    pltpu.async_copy(theta_ref.at[s16a], rows0, gsem0)
    _, s16b, _ = _indices(jnp.int32(1))
    pltpu.async_copy(theta_ref.at[s16b], rows1, gsem1)

    def _group(g, rows, gsem):
        m, s16, d16 = _indices(g)
        pltpu.make_async_copy(theta_ref.at[s16], rows, gsem).wait()
        a_s = plsc.load_gather(asrc_v, [s16])
        a_d = plsc.load_gather(adst_v, [d16])
        c_d = plsc.load_gather(cst_v, [d16])
        zz = a_s + a_d
        ex = jnp.where(m, jnp.exp(_lrelu(zz) - c_d), 0.0)
        for j in range(16):
            sj = ex[j]
            for cc in range(ROW // 16):
                sl = pl.ds(cc * 16, 16)
                rows[j, sl] = rows[j, sl] * sj
        dloc = d16 - rlo
        pltpu.async_copy(rows, acc.at[dloc], ssem, add=True).wait()
        _, s2, _ = _indices(g + 2)
        pltpu.async_copy(theta_ref.at[s2], rows, gsem)

    def _pair(i, carry):
        _group(2 * i, rows0, gsem0)
        _group(2 * i + 1, rows1, gsem1)
        return carry

    nt = (cnt + 31) // 32
    lax.fori_loop(0, nt, _pair, jnp.int32(0))

    _, sda, _ = _indices(2 * nt)
    pltpu.make_async_copy(theta_ref.at[sda], rows0, gsem0).wait()
    _, sdb, _ = _indices(2 * nt + 1)
    pltpu.make_async_copy(theta_ref.at[sdb], rows1, gsem1).wait()

    plsc.subcore_barrier()
    pltpu.sync_copy(acc.at[pl.ds(lo, NRT)],
                    out_ref.at[pl.ds(rlo + lo, NRT)])


def _sc_edge(theta_ext, asrc, adst, cst, srcp, dstp):
    mesh = plsc.VectorSubcoreMesh(core_axis_name="c", subcore_axis_name="s")
    return pl.kernel(
        _sc_edge_body,
        out_type=jax.ShapeDtypeStruct((NPAD, ROW), jnp.float32),
        mesh=mesh,
        scratch_types=[
            pltpu.VMEM((NPAD,), jnp.float32),
            pltpu.VMEM((NPAD,), jnp.float32),
            pltpu.VMEM((NPAD,), jnp.float32),
            pltpu.VMEM((ECH + 32,), jnp.int32),
            pltpu.VMEM((ECH + 32,), jnp.int32),
            pltpu.VMEM((16, ROW), jnp.float32),
            pltpu.VMEM((16, ROW), jnp.float32),
            pltpu.VMEM_SHARED((HALF, ROW), jnp.float32),
            pltpu.SemaphoreType.DMA,
            pltpu.SemaphoreType.DMA,
            pltpu.SemaphoreType.DMA,
        ],
        compiler_params=pltpu.CompilerParams(needs_layout_passes=False,
                                             use_tc_tiling_on_sc=False),
    )(theta_ext, asrc, adst, cst, srcp, dstp)


def kernel(x, edge_index, batch, W0, a_src0, a_dst0, b0, g0, be0, W1, a_src1,
           a_dst1, b1, g1, be1, dW1, db1, dW2, db2):
    xp = jnp.pad(x, ((0, NPAD - N), (0, 0)))
    loop = jnp.arange(N, dtype=jnp.int32)
    fill = jnp.full((EPAD - E - N,), N, jnp.int32)
    srcp = jnp.concatenate([edge_index[0], loop, fill])
    dstp = jnp.concatenate([edge_index[1], loop, fill])
    batch2d = jnp.concatenate([batch, jnp.full((NPAD - N,), G, jnp.int32)])
    batch2d = batch2d.reshape(NPAD, 1)

    asv0 = a_src0.reshape(1, C)
    adv0 = a_dst0.reshape(1, C)
    asv1 = a_src1.reshape(1, C)
    adv1 = a_dst1.reshape(1, C)
    b0r, g0r, be0r = (t.reshape(1, C) for t in (b0, g0, be0))
    b1r, g1r, be1r = (t.reshape(1, C) for t in (b1, g1, be1))
    db1r = db1.reshape(1, 2 * C)
    db2r = db2.reshape(1, C)

    ext0, as0, ad0, c0 = _tc_prep(xp, W0, asv0, adv0)
    acc0 = _sc_edge(ext0, as0.reshape(NPAD), ad0.reshape(NPAD),
                    c0.reshape(NPAD), srcp, dstp)
    ext1, as1, ad1, c1 = _tc_mid(acc0, b0r, g0r, be0r, W1, asv1, adv1)
    acc1 = _sc_edge(ext1, as1.reshape(NPAD), ad1.reshape(NPAD),
                    c1.reshape(NPAD), srcp, dstp)
    _, out = _tc_final(acc1, b1r, g1r, be1r, batch2d, dW1, db1r, dW2, db2r)
    return out


# speedup vs baseline: 36.4405x; 1.5467x over previous
import jax
import jax.numpy as jnp
from jax import lax
from jax.experimental import pallas as pl
from jax.experimental.pallas import tpu as pltpu
from jax.experimental.pallas import tpu_sc as plsc

N = 10000
E = 320000
D = 128
C = 128
G = 64

NPAD = 10240
HALF = NPAD // 2
ROW = 144
NS = 16
ECH = 20672
EPAD = NS * ECH
NG = ECH // 16
BLK = 1024
NRT = HALF // NS
K = 64
KG = K // 16
MAXB = ECH // K + 3
SVLEN = ECH + 3 * K + 16


def _lrelu(z):
    return jnp.maximum(z, 0.2 * z)


def _attn_heads(th, asv, adv, as_ref, ad_ref):
    as_ref[...] = jnp.sum(th * asv, axis=1, keepdims=True)
    ad_ref[...] = jnp.sum(th * adv, axis=1, keepdims=True)


def _ones_col(n):
    lane = lax.broadcasted_iota(jnp.int32, (n, ROW - C), 1)
    return jnp.where(lane == 0, 1.0, 0.0).astype(jnp.float32)


def _tc_prep_body(x_ref, w_ref, asv_ref, adv_ref, ext_ref, as_ref, ad_ref):
    th = jnp.dot(x_ref[...], w_ref[...], preferred_element_type=jnp.float32)
    _attn_heads(th, asv_ref[...], adv_ref[...], as_ref, ad_ref)
    ext_ref[...] = jnp.concatenate([th, _ones_col(BLK)], axis=1)


def _combine_ln(acc_ref, b_ref, g_ref, be_ref):
    s = acc_ref[...]
    o = s[:, :C] / (s[:, C:C + 1] + 1e-16) + b_ref[...]
    m = jnp.mean(o, axis=1, keepdims=True)
    v = jnp.mean((o - m) * (o - m), axis=1, keepdims=True)
    hn = (o - m) / jnp.sqrt(v + 1e-5) * g_ref[...] + be_ref[...]
    return jnp.maximum(hn, 0.0)


def _tc_mid_body(acc_ref, b_ref, g_ref, be_ref, w_ref, asv_ref, adv_ref,
                 ext_ref, as_ref, ad_ref):
    h = _combine_ln(acc_ref, b_ref, g_ref, be_ref)
    th = jnp.dot(h, w_ref[...], preferred_element_type=jnp.float32)
    _attn_heads(th, asv_ref[...], adv_ref[...], as_ref, ad_ref)
    ext_ref[...] = jnp.concatenate([th, _ones_col(BLK)], axis=1)


def _tc_final_body(acc_ref, b_ref, g_ref, be_ref, batch_ref, dw1_ref, db1_ref,
                   dw2_ref, db2_ref, pool_ref, out_ref):
    i = pl.program_id(0)
    h = _combine_ln(acc_ref, b_ref, g_ref, be_ref)
    ext = jnp.concatenate([h, _ones_col(BLK)], axis=1)
    gid = lax.broadcasted_iota(jnp.int32, (1, G), 1)
    oh = (batch_ref[...] == gid).astype(jnp.float32)
    part = lax.dot_general(oh, ext, (((0,), (0,)), ((), ())),
                           preferred_element_type=jnp.float32)

    @pl.when(i == 0)
    def _():
        pool_ref[...] = jnp.zeros_like(pool_ref)

    pool_ref[...] += part

    @pl.when(i == NPAD // BLK - 1)
    def _():
        p = pool_ref[...]
        pooled = p[:, :C] / jnp.maximum(p[:, C:C + 1], 1.0)
        t = jnp.dot(pooled, dw1_ref[...], preferred_element_type=jnp.float32)
        t = jnp.maximum(t + db1_ref[...], 0.0)
        out_ref[...] = jnp.dot(t, dw2_ref[...],
                               preferred_element_type=jnp.float32) + db2_ref[...]


def _tc_prep(xp, w, asv, adv):
    return pl.pallas_call(
        _tc_prep_body,
        grid=(NPAD // BLK,),
        in_specs=[
            pl.BlockSpec((BLK, D), lambda i: (i, 0)),
            pl.BlockSpec((D, C), lambda i: (0, 0)),
            pl.BlockSpec((1, C), lambda i: (0, 0)),
            pl.BlockSpec((1, C), lambda i: (0, 0)),
        ],
        out_specs=[
            pl.BlockSpec((BLK, ROW), lambda i: (i, 0)),
            pl.BlockSpec((BLK, 1), lambda i: (i, 0)),
            pl.BlockSpec((BLK, 1), lambda i: (i, 0)),
        ],
        out_shape=[
            jax.ShapeDtypeStruct((NPAD, ROW), jnp.float32),
            jax.ShapeDtypeStruct((NPAD, 1), jnp.float32),
            jax.ShapeDtypeStruct((NPAD, 1), jnp.float32),
        ],
    )(xp, w, asv, adv)


def _tc_mid(acc, b, g, be, w, asv, adv):
    return pl.pallas_call(
        _tc_mid_body,
        grid=(NPAD // BLK,),
        in_specs=[
            pl.BlockSpec((BLK, ROW), lambda i: (i, 0)),
            pl.BlockSpec((1, C), lambda i: (0, 0)),
            pl.BlockSpec((1, C), lambda i: (0, 0)),
            pl.BlockSpec((1, C), lambda i: (0, 0)),
            pl.BlockSpec((C, C), lambda i: (0, 0)),
            pl.BlockSpec((1, C), lambda i: (0, 0)),
            pl.BlockSpec((1, C), lambda i: (0, 0)),
        ],
        out_specs=[
            pl.BlockSpec((BLK, ROW), lambda i: (i, 0)),
            pl.BlockSpec((BLK, 1), lambda i: (i, 0)),
            pl.BlockSpec((BLK, 1), lambda i: (i, 0)),
        ],
        out_shape=[
            jax.ShapeDtypeStruct((NPAD, ROW), jnp.float32),
            jax.ShapeDtypeStruct((NPAD, 1), jnp.float32),
            jax.ShapeDtypeStruct((NPAD, 1), jnp.float32),
        ],
    )(acc, b, g, be, w, asv, adv)


def _tc_final(acc, b, g, be, batch2d, dw1, db1, dw2, db2):
    return pl.pallas_call(
        _tc_final_body,
        grid=(NPAD // BLK,),
        in_specs=[
            pl.BlockSpec((BLK, ROW), lambda i: (i, 0)),
            pl.BlockSpec((1, C), lambda i: (0, 0)),
            pl.BlockSpec((1, C), lambda i: (0, 0)),
            pl.BlockSpec((1, C), lambda i: (0, 0)),
            pl.BlockSpec((BLK, 1), lambda i: (i, 0)),
            pl.BlockSpec((C, 2 * C), lambda i: (0, 0)),
            pl.BlockSpec((1, 2 * C), lambda i: (0, 0)),
            pl.BlockSpec((2 * C, C), lambda i: (0, 0)),
            pl.BlockSpec((1, C), lambda i: (0, 0)),
        ],
        out_specs=[
            pl.BlockSpec((G, ROW), lambda i: (0, 0)),
            pl.BlockSpec((G, C), lambda i: (0, 0)),
        ],
        out_shape=[
            jax.ShapeDtypeStruct((G, ROW), jnp.float32),
            jax.ShapeDtypeStruct((G, C), jnp.float32),
        ],
    )(acc, b, g, be, batch2d, dw1, db1, dw2, db2)


def _sc_edge_body(theta_ref, asrc_ref, adst_ref, src_ref, dst_ref,
                  out_ref, asrc_v, adst_v, src_v, dst_v, rows0,
                  rows1, acc, gsem0, gsem1, ssem):
    cid = lax.axis_index("c")
    sid = lax.axis_index("s")
    ebase = sid * ECH
    rlo = cid * HALF

    pltpu.sync_copy(asrc_ref, asrc_v)
    pltpu.sync_copy(adst_ref, adst_v)
    pltpu.sync_copy(src_ref.at[pl.ds(ebase, ECH)], src_v.at[pl.ds(0, ECH)])
    pltpu.sync_copy(dst_ref.at[pl.ds(ebase, ECH)], dst_v.at[pl.ds(0, ECH)])
    z16i = jnp.zeros((16,), jnp.int32)
    for t in range((SVLEN - ECH) // 16):
        src_v[pl.ds(ECH + t * 16, 16)] = z16i

    zero = jnp.zeros((16,), jnp.float32)
    for j in range(K):
        for cc in range(ROW // 16):
            rows0[j, pl.ds(cc * 16, 16)] = zero
    lo = sid * NRT
    for k in range(NRT // K):
        pltpu.sync_copy(rows0, acc.at[pl.ds(lo + k * K, K)])
    plsc.subcore_barrier()

    def _compact(g, cnt):
        off = g * 16
        s16 = src_v[pl.ds(off, 16)]
        d16 = dst_v[pl.ds(off, 16)]
        keep = (d16 >= rlo) & (d16 < rlo + HALF)
        plsc.store_compressed(src_v.at[pl.ds(cnt, 16)], s16, mask=keep)
        plsc.store_compressed(dst_v.at[pl.ds(cnt, 16)], d16, mask=keep)
        return cnt + plsc.all_reduce_population_count(keep)[0]

    cnt = lax.fori_loop(0, NG, _compact, jnp.int32(0))

    lane = lax.iota(jnp.int32, 16)
    nbat = (cnt + K - 1) // K

    def _mkdloc(g, carry):
        off = g * 16
        d16 = dst_v[pl.ds(off, 16)]
        m = (off + lane) < cnt
        dst_v[pl.ds(off, 16)] = jnp.where(m, d16 - rlo, 0)
        return carry

    lax.fori_loop(0, (nbat + 2) * KG, _mkdloc, jnp.int32(0))

    def _gissue(b, rows, gsem):
        pltpu.async_copy(theta_ref.at[src_v.at[pl.ds(b * K, K)]], rows, gsem)

    def _gwait(b, rows, gsem):
        pltpu.make_async_copy(theta_ref.at[src_v.at[pl.ds(b * K, K)]],
                              rows, gsem).wait()

    _gissue(jnp.int32(0), rows0, gsem0)
    _gissue(jnp.int32(1), rows1, gsem1)

    def _dobatch(b, rows, gsem):
        _gwait(b, rows, gsem)
        base = b * K
        for sub in range(KG):
            off16 = base + sub * 16
            s16 = src_v[pl.ds(off16, 16)]
            dg16 = dst_v[pl.ds(off16, 16)] + rlo
            m = (off16 + lane) < cnt
            a_s = plsc.load_gather(asrc_v, [jnp.where(m, s16, 0)])
            a_d = plsc.load_gather(adst_v, [dg16])
            a_dd = plsc.load_gather(asrc_v, [dg16])
            c_d = _lrelu(a_dd + a_d)
            zz = a_s + a_d
            ex = jnp.where(m, jnp.exp(_lrelu(zz) - c_d), 0.0)
            for j in range(16):
                sj = ex[j]
                r = sub * 16 + j
                for cc in range(ROW // 16):
                    sl = pl.ds(cc * 16, 16)
                    rows[r, sl] = rows[r, sl] * sj
        pltpu.async_copy(rows, acc.at[dst_v.at[pl.ds(base, K)]],
                         ssem, add=True).wait()
        _gissue(b + 2, rows, gsem)

    def _pair(i, carry):
        _dobatch(2 * i, rows0, gsem0)
        _dobatch(2 * i + 1, rows1, gsem1)
        return carry

    nt = (nbat + 1) // 2
    lax.fori_loop(0, nt, _pair, jnp.int32(0))

    _gwait(2 * nt, rows0, gsem0)
    _gwait(2 * nt + 1, rows1, gsem1)

    plsc.subcore_barrier()
    pltpu.sync_copy(acc.at[pl.ds(lo, NRT)],
                    out_ref.at[pl.ds(rlo + lo, NRT)])


def _sc_edge(theta_ext, asrc, adst, srcp, dstp):
    mesh = plsc.VectorSubcoreMesh(core_axis_name="c", subcore_axis_name="s")
    return pl.kernel(
        _sc_edge_body,
        out_type=jax.ShapeDtypeStruct((NPAD, ROW), jnp.float32),
        mesh=mesh,
        scratch_types=[
            pltpu.VMEM((NPAD,), jnp.float32),
            pltpu.VMEM((NPAD,), jnp.float32),
            pltpu.VMEM((SVLEN,), jnp.int32),
            pltpu.VMEM((SVLEN,), jnp.int32),
            pltpu.VMEM((K, ROW), jnp.float32),
            pltpu.VMEM((K, ROW), jnp.float32),
            pltpu.VMEM_SHARED((HALF, ROW), jnp.float32),
            pltpu.SemaphoreType.DMA,
            pltpu.SemaphoreType.DMA,
            pltpu.SemaphoreType.DMA,
        ],
        compiler_params=pltpu.CompilerParams(needs_layout_passes=False,
                                             use_tc_tiling_on_sc=False),
    )(theta_ext, asrc, adst, srcp, dstp)


def kernel(x, edge_index, batch, W0, a_src0, a_dst0, b0, g0, be0, W1, a_src1,
           a_dst1, b1, g1, be1, dW1, db1, dW2, db2):
    xp = jnp.pad(x, ((0, NPAD - N), (0, 0)))
    loop = jnp.arange(N, dtype=jnp.int32)
    fill = jnp.full((EPAD - E - N,), N, jnp.int32)
    srcp = jnp.concatenate([edge_index[0], loop, fill])
    dstp = jnp.concatenate([edge_index[1], loop, fill])
    batch2d = jnp.concatenate([batch, jnp.full((NPAD - N,), G, jnp.int32)])
    batch2d = batch2d.reshape(NPAD, 1)

    asv0 = a_src0.reshape(1, C)
    adv0 = a_dst0.reshape(1, C)
    asv1 = a_src1.reshape(1, C)
    adv1 = a_dst1.reshape(1, C)
    b0r, g0r, be0r = (t.reshape(1, C) for t in (b0, g0, be0))
    b1r, g1r, be1r = (t.reshape(1, C) for t in (b1, g1, be1))
    db1r = db1.reshape(1, 2 * C)
    db2r = db2.reshape(1, C)

    ext0, as0, ad0 = _tc_prep(xp, W0, asv0, adv0)
    acc0 = _sc_edge(ext0, as0.reshape(NPAD), ad0.reshape(NPAD), srcp, dstp)
    ext1, as1, ad1 = _tc_mid(acc0, b0r, g0r, be0r, W1, asv1, adv1)
    acc1 = _sc_edge(ext1, as1.reshape(NPAD), ad1.reshape(NPAD), srcp, dstp)
    _, out = _tc_final(acc1, b1r, g1r, be1r, batch2d, dW1, db1r, dW2, db2r)
    return out
